# baseline (speedup 1.0000x reference)
import functools
import numpy as np
import jax
import jax.numpy as jnp
from jax import lax
from jax.experimental import pallas as pl
from jax.experimental.pallas import tpu as pltpu
from jax.experimental.pallas import tpu_sc as plsc

N = 10000
E = 320000
H = 128
B = 64
NSYS = 10
TFIDF = 5000
BN_EPS = 1e-5
LN_EPS = 1e-5
NELEM = float(N * H)

NSC = 2
NTILE = 16
NW = NSC * NTILE
EPT = E // NW
CH = 40
NCH = EPT // CH
RPT = 640

_BN_SCALE = 1.0 / np.sqrt(1.0 + BN_EPS)


def _row(v):
    return v.reshape(1, -1)


def _in_proj_body(x_ref, w_ref, b_ref, g_ref, bb_ref, o_ref):
    z = jnp.dot(x_ref[...], w_ref[...], preferred_element_type=jnp.float32)
    z = (z + b_ref[...]) * g_ref[...] + bb_ref[...]
    o_ref[...] = jnp.maximum(z, 0.0)


def _in_proj(x, w, b, g, bb):
    blk = 1000
    grid = N // blk
    return pl.pallas_call(
        _in_proj_body,
        grid=(grid,),
        in_specs=[
            pl.BlockSpec((blk, H), lambda i: (i, 0)),
            pl.BlockSpec((H, H), lambda i: (0, 0)),
            pl.BlockSpec((1, H), lambda i: (0, 0)),
            pl.BlockSpec((1, H), lambda i: (0, 0)),
            pl.BlockSpec((1, H), lambda i: (0, 0)),
        ],
        out_specs=pl.BlockSpec((blk, H), lambda i: (i, 0)),
        out_shape=jax.ShapeDtypeStruct((N, H), jnp.float32),
    )(x, w, b, g, bb)


def _edge_e_body(ea_ref, epw_ref, epb_ref, we_ref, be_ref, o_ref):
    a = jnp.dot(ea_ref[...], epw_ref[...], preferred_element_type=jnp.float32)
    a = jnp.maximum(a + epb_ref[...], 0.0)
    o_ref[...] = jnp.dot(a, we_ref[...], preferred_element_type=jnp.float32) + be_ref[...]


def _edge_e(eap, epw, epb, we, be):
    blk = 3200
    grid = E // blk
    return pl.pallas_call(
        _edge_e_body,
        grid=(grid,),
        in_specs=[
            pl.BlockSpec((blk, 8), lambda i: (i, 0)),
            pl.BlockSpec((8, H), lambda i: (0, 0)),
            pl.BlockSpec((1, H), lambda i: (0, 0)),
            pl.BlockSpec((H, H), lambda i: (0, 0)),
            pl.BlockSpec((1, H), lambda i: (0, 0)),
        ],
        out_specs=pl.BlockSpec((blk, H), lambda i: (i, 0)),
        out_shape=jax.ShapeDtypeStruct((E, H), jnp.float32),
    )(eap, epw, epb, we, be)


def _node_mlp_body(h_ref, a0_ref, a1_ref, eps_ref, w1_ref, b1_ref, g_ref,
                   bb_ref, w2_ref, b2_ref, t_ref, ps_ref):
    z = (1.0 + eps_ref[...]) * h_ref[...] + a0_ref[...] + a1_ref[...]
    z = jnp.dot(z, w1_ref[...], preferred_element_type=jnp.float32)
    z = jnp.maximum((z + b1_ref[...]) * g_ref[...] + bb_ref[...], 0.0)
    t = jnp.dot(z, w2_ref[...], preferred_element_type=jnp.float32) + b2_ref[...]
    t_ref[...] = t
    s0 = jnp.sum(t)
    s1 = jnp.sum(t * t)
    li = lax.broadcasted_iota(jnp.int32, (1, 1, H), 2)
    ps_ref[...] = jnp.where(li == 0, s0, jnp.where(li == 1, s1, 0.0))


def _node_mlp(h, a0, a1, epsb, w1, b1, g, bb, w2, b2):
    blk = 1000
    grid = N // blk
    return pl.pallas_call(
        _node_mlp_body,
        grid=(grid,),
        in_specs=[
            pl.BlockSpec((blk, H), lambda i: (i, 0)),
            pl.BlockSpec((blk, H), lambda i: (i, 0)),
            pl.BlockSpec((blk, H), lambda i: (N // 1000 + i, 0)),
            pl.BlockSpec((1, H), lambda i: (0, 0)),
            pl.BlockSpec((H, H), lambda i: (0, 0)),
            pl.BlockSpec((1, H), lambda i: (0, 0)),
            pl.BlockSpec((1, H), lambda i: (0, 0)),
            pl.BlockSpec((1, H), lambda i: (0, 0)),
            pl.BlockSpec((H, H), lambda i: (0, 0)),
            pl.BlockSpec((1, H), lambda i: (0, 0)),
        ],
        out_specs=[
            pl.BlockSpec((blk, H), lambda i: (i, 0)),
            pl.BlockSpec((1, 1, H), lambda i: (i, 0, 0)),
        ],
        out_shape=[
            jax.ShapeDtypeStruct((N, H), jnp.float32),
            jax.ShapeDtypeStruct((grid, 1, H), jnp.float32),
        ],
    )(h, a0, a1, epsb, w1, b1, g, bb, w2, b2)


def _ln_res_body(t_ref, h_ref, ps_ref, lw_ref, lb_ref, o_ref):
    psv = ps_ref[...]
    mu = jnp.sum(psv[:, :, 0:1]) / NELEM
    ms = jnp.sum(psv[:, :, 1:2]) / NELEM
    var = ms - mu * mu
    std = jnp.sqrt(var)
    y = (t_ref[...] - mu) / (std + LN_EPS) * lw_ref[...] + lb_ref[...]
    o_ref[...] = jnp.maximum(y + h_ref[...], 0.0)


def _ln_res(t, h, ps, lw, lb):
    blk = 1000
    grid = N // blk
    return pl.pallas_call(
        _ln_res_body,
        grid=(grid,),
        in_specs=[
            pl.BlockSpec((blk, H), lambda i: (i, 0)),
            pl.BlockSpec((blk, H), lambda i: (i, 0)),
            pl.BlockSpec((grid, 1, H), lambda i: (0, 0, 0)),
            pl.BlockSpec((1, H), lambda i: (0, 0)),
            pl.BlockSpec((1, H), lambda i: (0, 0)),
        ],
        out_specs=pl.BlockSpec((blk, H), lambda i: (i, 0)),
        out_shape=jax.ShapeDtypeStruct((N, H), jnp.float32),
    )(t, h, ps, lw, lb)


def _pool_body(smin_ref, smax_ref, h1_ref, h2_ref, h3_ref, bat_ref, batp_ref,
               w1_ref, b1_ref, w2_ref, sums_ref, cnt_ref, mx_ref):
    i = pl.program_id(0)
    hs = (h1_ref[...], h2_ref[...], h3_ref[...])
    scores = []
    for hb in hs:
        q = jnp.dot(hb, w1_ref[...], preferred_element_type=jnp.float32)
        q = jnp.maximum(q + b1_ref[...], 0.0)
        scores.append(jnp.sum(q * w2_ref[...], axis=1, keepdims=True))
    m = jnp.maximum(jnp.maximum(scores[0], scores[1]), scores[2])
    es = [jnp.exp(s - m) for s in scores]
    den = es[0] + es[1] + es[2]
    xj = (es[0] * hs[0] + es[1] * hs[1] + es[2] * hs[2]) / den

    brow = bat_ref[0]
    gi = lax.broadcasted_iota(jnp.int32, (B, brow.shape[1]), 0)
    onehot = (brow == gi).astype(jnp.float32)
    sp = jnp.dot(onehot, xj, preferred_element_type=jnp.float32)
    cp = jnp.broadcast_to(jnp.sum(onehot, axis=1, keepdims=True), (B, H))

    @pl.when(i == 0)
    def _():
        sums_ref[...] = sp
        cnt_ref[...] = cp
        mx_ref[...] = jnp.full((B, H), -3.0e38, jnp.float32)

    @pl.when(i != 0)
    def _():
        sums_ref[...] += sp
        cnt_ref[...] += cp

    batp = batp_ref[...]
    smin = smin_ref[i]
    smax = smax_ref[i]

    def gbody(g, carry):
        big = jnp.where(batp == g.astype(jnp.float32), xj, -3.0e38)
        bm = jnp.max(big, axis=0, keepdims=True)
        mx_ref[pl.ds(g, 1), :] = jnp.maximum(mx_ref[pl.ds(g, 1), :], bm)
        return carry

    lax.fori_loop(smin, smax + 1, gbody, 0)


def _pool(smin, smax, h1, h2, h3, bat3, batp, w1, b1, w2):
    blk = 1000
    grid = N // blk
    return pl.pallas_call(
        _pool_body,
        grid_spec=pltpu.PrefetchScalarGridSpec(
            num_scalar_prefetch=2,
            grid=(grid,),
            in_specs=[
                pl.BlockSpec((blk, H), lambda i, *_: (i, 0)),
                pl.BlockSpec((blk, H), lambda i, *_: (i, 0)),
                pl.BlockSpec((blk, H), lambda i, *_: (i, 0)),
                pl.BlockSpec((1, 1, blk), lambda i, *_: (i, 0, 0)),
                pl.BlockSpec((blk, H), lambda i, *_: (i, 0)),
                pl.BlockSpec((H, B), lambda i, *_: (0, 0)),
                pl.BlockSpec((1, B), lambda i, *_: (0, 0)),
                pl.BlockSpec((1, B), lambda i, *_: (0, 0)),
            ],
            out_specs=[
                pl.BlockSpec((B, H), lambda i, *_: (0, 0)),
                pl.BlockSpec((B, H), lambda i, *_: (0, 0)),
                pl.BlockSpec((B, H), lambda i, *_: (0, 0)),
            ],
        ),
        out_shape=[
            jax.ShapeDtypeStruct((B, H), jnp.float32),
            jax.ShapeDtypeStruct((B, H), jnp.float32),
            jax.ShapeDtypeStruct((B, H), jnp.float32),
        ],
    )(smin, smax, h1, h2, h3, bat3, batp, w1, b1, w2)


def _ln_rows(h, g, b):
    mu = jnp.mean(h, axis=-1, keepdims=True)
    var = jnp.mean((h - mu) ** 2, axis=-1, keepdims=True)
    return (h - mu) / jnp.sqrt(var + LN_EPS) * g + b


def _head_body(sums_ref, cnt_ref, mx_ref, gf_ref, gfw_ref, gfb_ref, tf_ref,
               tfw_ref, tfb_ref, tlg_ref, tlb_ref, plg_ref, plb_ref, c1w_ref,
               c1b_ref, cg_ref, cb_ref, c2w_ref, c2b_ref, o_ref):
    sums = sums_ref[...]
    cnt = cnt_ref[...]
    mean = sums / jnp.maximum(cnt, 1.0)
    mx = jnp.where(cnt > 0.0, mx_ref[...], 0.0)
    gf = jnp.dot(gf_ref[...], gfw_ref[...], preferred_element_type=jnp.float32) + gfb_ref[...]
    tf = jnp.dot(tf_ref[...], tfw_ref[...], preferred_element_type=jnp.float32) + tfb_ref[...]
    tf = jnp.maximum(_ln_rows(tf, tlg_ref[...], tlb_ref[...]), 0.0)
    comb = jnp.concatenate([mean, mx, sums, gf, tf], axis=1)
    comb = _ln_rows(comb, plg_ref[...], plb_ref[...])
    c = jnp.dot(comb, c1w_ref[...], preferred_element_type=jnp.float32)
    c = jnp.maximum((c + c1b_ref[...]) * cg_ref[...] + cb_ref[...], 0.0)
    lg = jnp.dot(c, c2w_ref[...], preferred_element_type=jnp.float32) + c2b_ref[...]
    o_ref[...] = jnp.broadcast_to(lg, (B, H))


def _head(sums, cnt, mx, gfp, gfw, gfb, tf, tfw, tfb, tlg, tlb, plg, plb,
          c1w, c1b, cg, cb, c2w, c2b):
    CD = 3 * H + H // 4 + H
    return pl.pallas_call(
        _head_body,
        in_specs=[
            pl.BlockSpec((B, H), lambda: (0, 0)),
            pl.BlockSpec((B, H), lambda: (0, 0)),
            pl.BlockSpec((B, H), lambda: (0, 0)),
            pl.BlockSpec((B, 16), lambda: (0, 0)),
            pl.BlockSpec((16, H // 4), lambda: (0, 0)),
            pl.BlockSpec((1, H // 4), lambda: (0, 0)),
            pl.BlockSpec((B, TFIDF), lambda: (0, 0)),
            pl.BlockSpec((TFIDF, H), lambda: (0, 0)),
            pl.BlockSpec((1, H), lambda: (0, 0)),
            pl.BlockSpec((1, H), lambda: (0, 0)),
            pl.BlockSpec((1, H), lambda: (0, 0)),
            pl.BlockSpec((1, CD), lambda: (0, 0)),
            pl.BlockSpec((1, CD), lambda: (0, 0)),
            pl.BlockSpec((CD, H), lambda: (0, 0)),
            pl.BlockSpec((1, H), lambda: (0, 0)),
            pl.BlockSpec((1, H), lambda: (0, 0)),
            pl.BlockSpec((1, H), lambda: (0, 0)),
            pl.BlockSpec((H, 1), lambda: (0, 0)),
            pl.BlockSpec((1, 1), lambda: (0, 0)),
        ],
        out_specs=pl.BlockSpec((B, H), lambda: (0, 0)),
        out_shape=jax.ShapeDtypeStruct((B, H), jnp.float32),
    )(sums, cnt, mx, gfp, gfw, gfb, tf, tfw, tfb, tlg, tlb, plg, plb,
      c1w, c1b, cg, cb, c2w, c2b)


def _sc_body(h_hbm, src_hbm, dst_hbm, e_hbm, z_hbm, out_hbm,
             srcall, dstall, rows0, eb0, rows1, eb1, mb, scb,
             agg, sem0, sem1, scsem):
    cid = lax.axis_index("c")
    sid = lax.axis_index("s")
    wid = sid * NSC + cid
    ebase = wid * EPT
    rbase = pl.multiple_of(
        jnp.minimum((sid * (N // NTILE)) // 8 * 8, N - RPT), 8)

    pltpu.sync_copy(z_hbm, agg.at[pl.ds(rbase, RPT)])
    pltpu.sync_copy(src_hbm.at[pl.ds(ebase, EPT)], srcall)
    pltpu.sync_copy(dst_hbm.at[pl.ds(ebase, EPT)], dstall)
    plsc.subcore_barrier()

    def issue_loads(c, rows_b, e_b, sem):
        off = pl.multiple_of(c * CH, 8)
        pltpu.async_copy(h_hbm.at[srcall.at[pl.ds(off, CH)]], rows_b, sem)
        eoff = pl.multiple_of(ebase + c * CH, 8)
        pltpu.async_copy(e_hbm.at[pl.ds(eoff, CH)], e_b, sem)

    def wait_loads(c, rows_b, e_b, sem):
        off = pl.multiple_of(c * CH, 8)
        pltpu.make_async_copy(h_hbm.at[srcall.at[pl.ds(off, CH)]], rows_b, sem).wait()
        eoff = pl.multiple_of(ebase + c * CH, 8)
        pltpu.make_async_copy(e_hbm.at[pl.ds(eoff, CH)], e_b, sem).wait()

    def scatter_desc():
        return pltpu.make_async_copy(mb, agg.at[scb], scsem)

    def process(c, rows_b, e_b, sem):
        wait_loads(c, rows_b, e_b, sem)

        @pl.when(c >= 1)
        def _():
            scatter_desc().wait()

        @plsc.parallel_loop(0, CH, unroll=4)
        def _(r):
            for k in range(H // 16):
                sl = pl.ds(k * 16, 16)
                mb[r, sl] = jnp.maximum(rows_b[r, sl] + e_b[r, sl], 0.0)

        for off in (0, 16, 24):
            doff = pl.multiple_of(c * CH + off, 8)
            scb[pl.ds(off, 16)] = dstall[pl.ds(doff, 16)]

        pltpu.async_copy(mb, agg.at[scb], scsem, add=True)

        @pl.when(c + 2 < NCH)
        def _():
            issue_loads(c + 2, rows_b, e_b, sem)

    issue_loads(0, rows0, eb0, sem0)
    issue_loads(1, rows1, eb1, sem1)

    def loop_body(g, carry):
        c0 = g * 2
        process(c0, rows0, eb0, sem0)
        process(c0 + 1, rows1, eb1, sem1)
        return carry

    lax.fori_loop(0, NCH // 2, loop_body, 0)
    scatter_desc().wait()

    plsc.subcore_barrier()
    obase = pl.multiple_of(cid * N + rbase, 8)
    pltpu.sync_copy(agg.at[pl.ds(rbase, RPT)], out_hbm.at[pl.ds(obase, RPT)])


_sc_msg_cache = []


def _sc_msg(h, src, dst, e, zeros):
    if not _sc_msg_cache:
        _sc_msg_cache.append(functools.partial(
            pl.kernel,
            out_type=jax.ShapeDtypeStruct((2 * N, H), jnp.float32),
            mesh=plsc.VectorSubcoreMesh(core_axis_name="c", subcore_axis_name="s"),
            scratch_types=[
                pltpu.VMEM((EPT,), jnp.int32),
                pltpu.VMEM((EPT,), jnp.int32),
                pltpu.VMEM((CH, H), jnp.float32),
                pltpu.VMEM((CH, H), jnp.float32),
                pltpu.VMEM((CH, H), jnp.float32),
                pltpu.VMEM((CH, H), jnp.float32),
                pltpu.VMEM((CH, H), jnp.float32),
                pltpu.VMEM((CH,), jnp.int32),
                pltpu.VMEM_SHARED((N, H), jnp.float32),
                pltpu.SemaphoreType.DMA,
                pltpu.SemaphoreType.DMA,
                pltpu.SemaphoreType.DMA,
            ],
        )(_sc_body))
    return _sc_msg_cache[0](h, src, dst, e, zeros)


def kernel(x, edge_index, edge_attr, batch, graph_features, tfidf_features, params):
    p = params
    src = edge_index[0].astype(jnp.int32)
    dst = edge_index[1].astype(jnp.int32)
    eap = jnp.pad(edge_attr, ((0, 0), (0, 2)))
    epw = jnp.pad(p['ep_W'], ((0, 2), (0, 0)))
    zeros = jnp.zeros((RPT, H), jnp.float32)

    h = _in_proj(x, p['in_W'], _row(p['in_b']),
                 _row(p['in_bng'] * _BN_SCALE), _row(p['in_bnb']))

    es = [_edge_e(eap, epw, _row(p['ep_b']), lp['We'], _row(lp['be']))
          for lp in p['layers']]

    hs = []
    for li, lp in enumerate(p['layers']):
        agg2 = _sc_msg(h, src, dst, es[li], zeros)
        epsb = jnp.broadcast_to(lp['eps'].reshape(1, 1), (1, H))
        t, ps = _node_mlp(h, agg2, agg2, epsb, lp['W1'], _row(lp['b1']),
                          _row(lp['bng'] * _BN_SCALE), _row(lp['bnb']),
                          lp['W2'], _row(lp['b2']))
        h = _ln_res(t, h, ps, _row(lp['lnw']), _row(lp['lnb']))
        hs.append(h)

    bat3 = batch.astype(jnp.int32).reshape(N // 1000, 1, 1000)
    batp = jnp.broadcast_to(batch.astype(jnp.float32)[:, None], (N, H))
    bi = batch.astype(jnp.int32)
    smin = bi[0::1000]
    smax = bi[999::1000]
    sums, cnt, mx = _pool(smin, smax, hs[0], hs[1], hs[2], bat3, batp,
                          p['jk_W1'], _row(p['jk_b1']), _row(p['jk_W2'][:, 0]))

    gfp = jnp.pad(graph_features, ((0, 0), (0, 16 - NSYS)))
    gfw = jnp.pad(p['gf_W'], ((0, 16 - NSYS), (0, 0)))
    out = _head(sums, cnt, mx, gfp, gfw, _row(p['gf_b']),
                tfidf_features, p['tf_W'], _row(p['tf_b']),
                _row(p['tf_lng']), _row(p['tf_lnb']),
                _row(p['pre_lng']), _row(p['pre_lnb']),
                p['c1_W'], _row(p['c1_b']),
                _row(p['c_bng'] * _BN_SCALE), _row(p['c_bnb']),
                p['c2_W'], p['c2_b'].reshape(1, 1))
    return out[:, 0]


# speedup vs baseline: 1.0710x; 1.0710x over previous
import functools
import numpy as np
import jax
import jax.numpy as jnp
from jax import lax
from jax.experimental import pallas as pl
from jax.experimental.pallas import tpu as pltpu
from jax.experimental.pallas import tpu_sc as plsc

N = 10000
E = 320000
H = 128
B = 64
NSYS = 10
TFIDF = 5000
BN_EPS = 1e-5
LN_EPS = 1e-5
NELEM = float(N * H)

NSC = 2
NTILE = 16
NW = NSC * NTILE
EPT = E // NW
CH = 40
NCH = EPT // CH
RPT = 640

_BN_SCALE = 1.0 / np.sqrt(1.0 + BN_EPS)


def _row(v):
    return v.reshape(1, -1)


def _in_proj_body(x_ref, w_ref, b_ref, g_ref, bb_ref, o_ref):
    z = jnp.dot(x_ref[...], w_ref[...], preferred_element_type=jnp.float32)
    z = (z + b_ref[...]) * g_ref[...] + bb_ref[...]
    o_ref[...] = jnp.maximum(z, 0.0)


def _in_proj(x, w, b, g, bb):
    blk = 1000
    grid = N // blk
    return pl.pallas_call(
        _in_proj_body,
        grid=(grid,),
        in_specs=[
            pl.BlockSpec((blk, H), lambda i: (i, 0)),
            pl.BlockSpec((H, H), lambda i: (0, 0)),
            pl.BlockSpec((1, H), lambda i: (0, 0)),
            pl.BlockSpec((1, H), lambda i: (0, 0)),
            pl.BlockSpec((1, H), lambda i: (0, 0)),
        ],
        out_specs=pl.BlockSpec((blk, H), lambda i: (i, 0)),
        out_shape=jax.ShapeDtypeStruct((N, H), jnp.float32),
    )(x, w, b, g, bb)


def _edge_e_body(ea_ref, epw_ref, epb_ref, we_ref, be_ref, o_ref):
    a = jnp.dot(ea_ref[...], epw_ref[...], preferred_element_type=jnp.float32)
    a = jnp.maximum(a + epb_ref[...], 0.0)
    o_ref[...] = jnp.dot(a, we_ref[...], preferred_element_type=jnp.float32) + be_ref[...]


def _edge_e(eap, epw, epb, we, be):
    blk = 3200
    grid = E // blk
    return pl.pallas_call(
        _edge_e_body,
        grid=(grid,),
        in_specs=[
            pl.BlockSpec((blk, 8), lambda i: (i, 0)),
            pl.BlockSpec((8, H), lambda i: (0, 0)),
            pl.BlockSpec((1, H), lambda i: (0, 0)),
            pl.BlockSpec((H, H), lambda i: (0, 0)),
            pl.BlockSpec((1, H), lambda i: (0, 0)),
        ],
        out_specs=pl.BlockSpec((blk, H), lambda i: (i, 0)),
        out_shape=jax.ShapeDtypeStruct((E, H), jnp.float32),
    )(eap, epw, epb, we, be)


def _node_mlp_body(h_ref, a0_ref, a1_ref, eps_ref, w1_ref, b1_ref, g_ref,
                   bb_ref, w2_ref, b2_ref, t_ref, ps_ref):
    z = (1.0 + eps_ref[...]) * h_ref[...] + a0_ref[...] + a1_ref[...]
    z = jnp.dot(z, w1_ref[...], preferred_element_type=jnp.float32)
    z = jnp.maximum((z + b1_ref[...]) * g_ref[...] + bb_ref[...], 0.0)
    t = jnp.dot(z, w2_ref[...], preferred_element_type=jnp.float32) + b2_ref[...]
    t_ref[...] = t
    s0 = jnp.sum(t)
    s1 = jnp.sum(t * t)
    li = lax.broadcasted_iota(jnp.int32, (1, 1, H), 2)
    ps_ref[...] = jnp.where(li == 0, s0, jnp.where(li == 1, s1, 0.0))


def _node_mlp(h, a0, a1, epsb, w1, b1, g, bb, w2, b2):
    blk = 1000
    grid = N // blk
    return pl.pallas_call(
        _node_mlp_body,
        grid=(grid,),
        in_specs=[
            pl.BlockSpec((blk, H), lambda i: (i, 0)),
            pl.BlockSpec((blk, H), lambda i: (i, 0)),
            pl.BlockSpec((blk, H), lambda i: (N // 1000 + i, 0)),
            pl.BlockSpec((1, H), lambda i: (0, 0)),
            pl.BlockSpec((H, H), lambda i: (0, 0)),
            pl.BlockSpec((1, H), lambda i: (0, 0)),
            pl.BlockSpec((1, H), lambda i: (0, 0)),
            pl.BlockSpec((1, H), lambda i: (0, 0)),
            pl.BlockSpec((H, H), lambda i: (0, 0)),
            pl.BlockSpec((1, H), lambda i: (0, 0)),
        ],
        out_specs=[
            pl.BlockSpec((blk, H), lambda i: (i, 0)),
            pl.BlockSpec((1, 1, H), lambda i: (i, 0, 0)),
        ],
        out_shape=[
            jax.ShapeDtypeStruct((N, H), jnp.float32),
            jax.ShapeDtypeStruct((grid, 1, H), jnp.float32),
        ],
    )(h, a0, a1, epsb, w1, b1, g, bb, w2, b2)


def _ln_res_body(t_ref, h_ref, ps_ref, lw_ref, lb_ref, o_ref):
    psv = ps_ref[...]
    mu = jnp.sum(psv[:, :, 0:1]) / NELEM
    ms = jnp.sum(psv[:, :, 1:2]) / NELEM
    var = ms - mu * mu
    std = jnp.sqrt(var)
    y = (t_ref[...] - mu) / (std + LN_EPS) * lw_ref[...] + lb_ref[...]
    o_ref[...] = jnp.maximum(y + h_ref[...], 0.0)


def _ln_res(t, h, ps, lw, lb):
    blk = 1000
    grid = N // blk
    return pl.pallas_call(
        _ln_res_body,
        grid=(grid,),
        in_specs=[
            pl.BlockSpec((blk, H), lambda i: (i, 0)),
            pl.BlockSpec((blk, H), lambda i: (i, 0)),
            pl.BlockSpec((grid, 1, H), lambda i: (0, 0, 0)),
            pl.BlockSpec((1, H), lambda i: (0, 0)),
            pl.BlockSpec((1, H), lambda i: (0, 0)),
        ],
        out_specs=pl.BlockSpec((blk, H), lambda i: (i, 0)),
        out_shape=jax.ShapeDtypeStruct((N, H), jnp.float32),
    )(t, h, ps, lw, lb)


def _pool_body(smin_ref, smax_ref, h1_ref, h2_ref, h3_ref, bat_ref, batp_ref,
               w1_ref, b1_ref, w2_ref, sums_ref, cnt_ref, mx_ref):
    i = pl.program_id(0)
    hs = (h1_ref[...], h2_ref[...], h3_ref[...])
    scores = []
    for hb in hs:
        q = jnp.dot(hb, w1_ref[...], preferred_element_type=jnp.float32)
        q = jnp.maximum(q + b1_ref[...], 0.0)
        scores.append(jnp.sum(q * w2_ref[...], axis=1, keepdims=True))
    m = jnp.maximum(jnp.maximum(scores[0], scores[1]), scores[2])
    es = [jnp.exp(s - m) for s in scores]
    den = es[0] + es[1] + es[2]
    xj = (es[0] * hs[0] + es[1] * hs[1] + es[2] * hs[2]) / den

    brow = bat_ref[0]
    gi = lax.broadcasted_iota(jnp.int32, (B, brow.shape[1]), 0)
    onehot = (brow == gi).astype(jnp.float32)
    sp = jnp.dot(onehot, xj, preferred_element_type=jnp.float32)
    cp = jnp.broadcast_to(jnp.sum(onehot, axis=1, keepdims=True), (B, H))

    @pl.when(i == 0)
    def _():
        sums_ref[...] = sp
        cnt_ref[...] = cp
        mx_ref[...] = jnp.full((B, H), -3.0e38, jnp.float32)

    @pl.when(i != 0)
    def _():
        sums_ref[...] += sp
        cnt_ref[...] += cp

    batp = batp_ref[...]
    smin = smin_ref[i]
    smax = smax_ref[i]

    def gbody(g, carry):
        big = jnp.where(batp == g.astype(jnp.float32), xj, -3.0e38)
        bm = jnp.max(big, axis=0, keepdims=True)
        mx_ref[pl.ds(g, 1), :] = jnp.maximum(mx_ref[pl.ds(g, 1), :], bm)
        return carry

    lax.fori_loop(smin, smax + 1, gbody, 0)


def _pool(smin, smax, h1, h2, h3, bat3, batp, w1, b1, w2):
    blk = 1000
    grid = N // blk
    return pl.pallas_call(
        _pool_body,
        grid_spec=pltpu.PrefetchScalarGridSpec(
            num_scalar_prefetch=2,
            grid=(grid,),
            in_specs=[
                pl.BlockSpec((blk, H), lambda i, *_: (i, 0)),
                pl.BlockSpec((blk, H), lambda i, *_: (i, 0)),
                pl.BlockSpec((blk, H), lambda i, *_: (i, 0)),
                pl.BlockSpec((1, 1, blk), lambda i, *_: (i, 0, 0)),
                pl.BlockSpec((blk, H), lambda i, *_: (i, 0)),
                pl.BlockSpec((H, B), lambda i, *_: (0, 0)),
                pl.BlockSpec((1, B), lambda i, *_: (0, 0)),
                pl.BlockSpec((1, B), lambda i, *_: (0, 0)),
            ],
            out_specs=[
                pl.BlockSpec((B, H), lambda i, *_: (0, 0)),
                pl.BlockSpec((B, H), lambda i, *_: (0, 0)),
                pl.BlockSpec((B, H), lambda i, *_: (0, 0)),
            ],
        ),
        out_shape=[
            jax.ShapeDtypeStruct((B, H), jnp.float32),
            jax.ShapeDtypeStruct((B, H), jnp.float32),
            jax.ShapeDtypeStruct((B, H), jnp.float32),
        ],
    )(smin, smax, h1, h2, h3, bat3, batp, w1, b1, w2)


def _ln_rows(h, g, b):
    mu = jnp.mean(h, axis=-1, keepdims=True)
    var = jnp.mean((h - mu) ** 2, axis=-1, keepdims=True)
    return (h - mu) / jnp.sqrt(var + LN_EPS) * g + b


def _head_body(sums_ref, cnt_ref, mx_ref, gf_ref, gfw_ref, gfb_ref, tf_ref,
               tfw_ref, tfb_ref, tlg_ref, tlb_ref, plg_ref, plb_ref, c1w_ref,
               c1b_ref, cg_ref, cb_ref, c2w_ref, c2b_ref, o_ref):
    sums = sums_ref[...]
    cnt = cnt_ref[...]
    mean = sums / jnp.maximum(cnt, 1.0)
    mx = jnp.where(cnt > 0.0, mx_ref[...], 0.0)
    gf = jnp.dot(gf_ref[...], gfw_ref[...], preferred_element_type=jnp.float32) + gfb_ref[...]
    tf = jnp.dot(tf_ref[...], tfw_ref[...], preferred_element_type=jnp.float32) + tfb_ref[...]
    tf = jnp.maximum(_ln_rows(tf, tlg_ref[...], tlb_ref[...]), 0.0)
    comb = jnp.concatenate([mean, mx, sums, gf, tf], axis=1)
    comb = _ln_rows(comb, plg_ref[...], plb_ref[...])
    c = jnp.dot(comb, c1w_ref[...], preferred_element_type=jnp.float32)
    c = jnp.maximum((c + c1b_ref[...]) * cg_ref[...] + cb_ref[...], 0.0)
    lg = jnp.dot(c, c2w_ref[...], preferred_element_type=jnp.float32) + c2b_ref[...]
    o_ref[...] = jnp.broadcast_to(lg, (B, H))


def _head(sums, cnt, mx, gfp, gfw, gfb, tf, tfw, tfb, tlg, tlb, plg, plb,
          c1w, c1b, cg, cb, c2w, c2b):
    CD = 3 * H + H // 4 + H
    return pl.pallas_call(
        _head_body,
        in_specs=[
            pl.BlockSpec((B, H), lambda: (0, 0)),
            pl.BlockSpec((B, H), lambda: (0, 0)),
            pl.BlockSpec((B, H), lambda: (0, 0)),
            pl.BlockSpec((B, 16), lambda: (0, 0)),
            pl.BlockSpec((16, H // 4), lambda: (0, 0)),
            pl.BlockSpec((1, H // 4), lambda: (0, 0)),
            pl.BlockSpec((B, TFIDF), lambda: (0, 0)),
            pl.BlockSpec((TFIDF, H), lambda: (0, 0)),
            pl.BlockSpec((1, H), lambda: (0, 0)),
            pl.BlockSpec((1, H), lambda: (0, 0)),
            pl.BlockSpec((1, H), lambda: (0, 0)),
            pl.BlockSpec((1, CD), lambda: (0, 0)),
            pl.BlockSpec((1, CD), lambda: (0, 0)),
            pl.BlockSpec((CD, H), lambda: (0, 0)),
            pl.BlockSpec((1, H), lambda: (0, 0)),
            pl.BlockSpec((1, H), lambda: (0, 0)),
            pl.BlockSpec((1, H), lambda: (0, 0)),
            pl.BlockSpec((H, 1), lambda: (0, 0)),
            pl.BlockSpec((1, 1), lambda: (0, 0)),
        ],
        out_specs=pl.BlockSpec((B, H), lambda: (0, 0)),
        out_shape=jax.ShapeDtypeStruct((B, H), jnp.float32),
    )(sums, cnt, mx, gfp, gfw, gfb, tf, tfw, tfb, tlg, tlb, plg, plb,
      c1w, c1b, cg, cb, c2w, c2b)


def _sc_body(h_hbm, src_hbm, dst_hbm, e_hbm, z_hbm, out_hbm,
             srcall, rows0, eb0, db0, rows1, eb1, db1, rows2, eb2, db2,
             mb, scb, agg, sem0, sem1, sem2, scsem):
    cid = lax.axis_index("c")
    sid = lax.axis_index("s")
    wid = sid * NSC + cid
    ebase = wid * EPT
    rbase = pl.multiple_of(
        jnp.minimum((sid * (N // NTILE)) // 8 * 8, N - RPT), 8)

    pltpu.sync_copy(z_hbm, agg.at[pl.ds(rbase, RPT)])
    pltpu.sync_copy(src_hbm.at[pl.ds(ebase, EPT)], srcall)
    plsc.subcore_barrier()

    def issue_loads(c, rows_b, e_b, d_b, sem):
        off = pl.multiple_of(c * CH, 8)
        pltpu.async_copy(h_hbm.at[srcall.at[pl.ds(off, CH)]], rows_b, sem)
        eoff = pl.multiple_of(ebase + c * CH, 8)
        pltpu.async_copy(e_hbm.at[pl.ds(eoff, CH)], e_b, sem)
        pltpu.async_copy(dst_hbm.at[pl.ds(eoff, CH)], d_b, sem)

    def wait_loads(c, rows_b, e_b, d_b, sem):
        off = pl.multiple_of(c * CH, 8)
        pltpu.make_async_copy(h_hbm.at[srcall.at[pl.ds(off, CH)]], rows_b, sem).wait()
        eoff = pl.multiple_of(ebase + c * CH, 8)
        pltpu.make_async_copy(e_hbm.at[pl.ds(eoff, CH)], e_b, sem).wait()
        pltpu.make_async_copy(dst_hbm.at[pl.ds(eoff, CH)], d_b, sem).wait()

    def scatter_desc():
        return pltpu.make_async_copy(mb, agg.at[scb], scsem)

    def process(c, rows_b, e_b, d_b, sem):
        wait_loads(c, rows_b, e_b, d_b, sem)

        @pl.when(c >= 1)
        def _():
            scatter_desc().wait()

        @plsc.parallel_loop(0, CH, unroll=4)
        def _(r):
            for k in range(H // 16):
                sl = pl.ds(k * 16, 16)
                mb[r, sl] = jnp.maximum(rows_b[r, sl] + e_b[r, sl], 0.0)

        for off in (0, 16, 24):
            scb[pl.ds(off, 16)] = d_b[pl.ds(off, 16)]

        pltpu.async_copy(mb, agg.at[scb], scsem, add=True)

        @pl.when(c + 3 < NCH)
        def _():
            issue_loads(c + 3, rows_b, e_b, d_b, sem)

    issue_loads(0, rows0, eb0, db0, sem0)
    issue_loads(1, rows1, eb1, db1, sem1)
    issue_loads(2, rows2, eb2, db2, sem2)

    def loop_body(g, carry):
        c0 = g * 3
        process(c0, rows0, eb0, db0, sem0)
        process(c0 + 1, rows1, eb1, db1, sem1)
        process(c0 + 2, rows2, eb2, db2, sem2)
        return carry

    lax.fori_loop(0, NCH // 3, loop_body, 0)
    process(NCH - 1, rows0, eb0, db0, sem0)
    scatter_desc().wait()

    plsc.subcore_barrier()
    obase = pl.multiple_of(cid * N + rbase, 8)
    pltpu.sync_copy(agg.at[pl.ds(rbase, RPT)], out_hbm.at[pl.ds(obase, RPT)])


_sc_msg_cache = []


def _sc_msg(h, src, dst, e, zeros):
    if not _sc_msg_cache:
        _sc_msg_cache.append(functools.partial(
            pl.kernel,
            out_type=jax.ShapeDtypeStruct((2 * N, H), jnp.float32),
            mesh=plsc.VectorSubcoreMesh(core_axis_name="c", subcore_axis_name="s"),
            scratch_types=[
                pltpu.VMEM((EPT,), jnp.int32),
                pltpu.VMEM((CH, H), jnp.float32),
                pltpu.VMEM((CH, H), jnp.float32),
                pltpu.VMEM((CH,), jnp.int32),
                pltpu.VMEM((CH, H), jnp.float32),
                pltpu.VMEM((CH, H), jnp.float32),
                pltpu.VMEM((CH,), jnp.int32),
                pltpu.VMEM((CH, H), jnp.float32),
                pltpu.VMEM((CH, H), jnp.float32),
                pltpu.VMEM((CH,), jnp.int32),
                pltpu.VMEM((CH, H), jnp.float32),
                pltpu.VMEM((CH,), jnp.int32),
                pltpu.VMEM_SHARED((N, H), jnp.float32),
                pltpu.SemaphoreType.DMA,
                pltpu.SemaphoreType.DMA,
                pltpu.SemaphoreType.DMA,
                pltpu.SemaphoreType.DMA,
            ],
        )(_sc_body))
    return _sc_msg_cache[0](h, src, dst, e, zeros)


def kernel(x, edge_index, edge_attr, batch, graph_features, tfidf_features, params):
    p = params
    src = edge_index[0].astype(jnp.int32)
    dst = edge_index[1].astype(jnp.int32)
    eap = jnp.pad(edge_attr, ((0, 0), (0, 2)))
    epw = jnp.pad(p['ep_W'], ((0, 2), (0, 0)))
    zeros = jnp.zeros((RPT, H), jnp.float32)

    h = _in_proj(x, p['in_W'], _row(p['in_b']),
                 _row(p['in_bng'] * _BN_SCALE), _row(p['in_bnb']))

    es = [_edge_e(eap, epw, _row(p['ep_b']), lp['We'], _row(lp['be']))
          for lp in p['layers']]

    hs = []
    for li, lp in enumerate(p['layers']):
        agg2 = _sc_msg(h, src, dst, es[li], zeros)
        epsb = jnp.broadcast_to(lp['eps'].reshape(1, 1), (1, H))
        t, ps = _node_mlp(h, agg2, agg2, epsb, lp['W1'], _row(lp['b1']),
                          _row(lp['bng'] * _BN_SCALE), _row(lp['bnb']),
                          lp['W2'], _row(lp['b2']))
        h = _ln_res(t, h, ps, _row(lp['lnw']), _row(lp['lnb']))
        hs.append(h)

    bat3 = batch.astype(jnp.int32).reshape(N // 1000, 1, 1000)
    batp = jnp.broadcast_to(batch.astype(jnp.float32)[:, None], (N, H))
    bi = batch.astype(jnp.int32)
    smin = bi[0::1000]
    smax = bi[999::1000]
    sums, cnt, mx = _pool(smin, smax, hs[0], hs[1], hs[2], bat3, batp,
                          p['jk_W1'], _row(p['jk_b1']), _row(p['jk_W2'][:, 0]))

    gfp = jnp.pad(graph_features, ((0, 0), (0, 16 - NSYS)))
    gfw = jnp.pad(p['gf_W'], ((0, 16 - NSYS), (0, 0)))
    out = _head(sums, cnt, mx, gfp, gfw, _row(p['gf_b']),
                tfidf_features, p['tf_W'], _row(p['tf_b']),
                _row(p['tf_lng']), _row(p['tf_lnb']),
                _row(p['pre_lng']), _row(p['pre_lnb']),
                p['c1_W'], _row(p['c1_b']),
                _row(p['c_bng'] * _BN_SCALE), _row(p['c_bnb']),
                p['c2_W'], p['c2_b'].reshape(1, 1))
    return out[:, 0]
